# baseline (speedup 1.0000x reference)
import functools

import jax
import jax.numpy as jnp
from jax import lax
from jax.experimental import pallas as pl
from jax.experimental.pallas import tpu as pltpu
from jax.experimental.pallas import tpu_sc as plsc

NUM_SC = 2
SUBCORES = 16
NUM_TILES = NUM_SC * SUBCORES
CHUNK = 64
DEPTH = 5
IDEPTH = 10
SDEPTH = 2
LANES = 16
PAD_ROWS = 16
BN_EPS_CONST = 1e-5


def _sc_aggregate(h, ei, n_full, n_extra):
    NP, D = h.shape
    step_rows = -(-NP // (SUBCORES * 8)) * 8
    last_rows = NP - (SUBCORES - 1) * step_rows
    assert last_rows > 0 and last_rows % 8 == 0
    n_loop = -(-(n_full + (1 if n_extra else 0)) // IDEPTH) * IDEPTH
    mesh = plsc.VectorSubcoreMesh(
        core_axis_name="c", subcore_axis_name="s",
        num_cores=NUM_SC, num_subcores=SUBCORES)

    @functools.partial(
        pl.kernel,
        out_type=jax.ShapeDtypeStruct((NUM_SC, NP, D), jnp.float32),
        mesh=mesh,
        scratch_types=[
            [pltpu.VMEM((2, CHUNK), jnp.int32) for _ in range(IDEPTH)],
            [pltpu.VMEM((CHUNK, D), jnp.float32) for _ in range(DEPTH)],
            pltpu.VMEM_SHARED((NP, D), jnp.float32),
            [pltpu.SemaphoreType.DMA for _ in range(IDEPTH)],
            [pltpu.SemaphoreType.DMA for _ in range(DEPTH)],
            [pltpu.SemaphoreType.DMA for _ in range(DEPTH)],
        ],
    )
    def agg_kernel(h_hbm, ei_hbm, out_hbm, islots, rows, acc,
                   isems, gsems, ssems):
        cid = lax.axis_index("c")
        sid = lax.axis_index("s")
        wid = sid * NUM_SC + cid
        n_mine = n_full + jnp.where(wid < n_extra, 1, 0)

        def _guard(k, lo_ok=True):
            return (k < n_mine) if lo_ok else ((k >= 0) & (k < n_mine))

        def fire_idx(k, si):
            @pl.when(_guard(k))
            def _():
                off = (k * NUM_TILES + wid) * CHUNK
                pltpu.async_copy(ei_hbm.at[0, pl.ds(off, CHUNK)],
                                 islots[si].at[0], isems[si])
                pltpu.async_copy(ei_hbm.at[1, pl.ds(off, CHUNK)],
                                 islots[si].at[1], isems[si])

        def wait_idx(k, si):
            @pl.when(_guard(k))
            def _():
                pltpu.make_async_copy(ei_hbm.at[0, pl.ds(0, CHUNK)],
                                      islots[si].at[0], isems[si]).wait()
                pltpu.make_async_copy(ei_hbm.at[1, pl.ds(0, CHUNK)],
                                      islots[si].at[1], isems[si]).wait()

        def fire_gather(k, s, si):
            @pl.when(_guard(k))
            def _():
                pltpu.async_copy(h_hbm.at[islots[si].at[0]], rows[s],
                                 gsems[s])

        def wait_gather(k, s, si):
            @pl.when(_guard(k))
            def _():
                pltpu.make_async_copy(
                    h_hbm.at[islots[si].at[0]], rows[s], gsems[s]).wait()

        def fire_scatter(k, s, si):
            @pl.when(_guard(k))
            def _():
                pltpu.async_copy(rows[s], acc.at[islots[si].at[1]],
                                 ssems[s], add=True)

        def wait_scatter(k, s, si):
            @pl.when(_guard(k, lo_ok=False))
            def _():
                pltpu.make_async_copy(rows[s], acc.at[islots[si].at[1]],
                                      ssems[s]).wait()

        for j in range(IDEPTH - SDEPTH):
            fire_idx(j, j)
        for j in range(DEPTH - SDEPTH):
            wait_idx(j, j)
            fire_gather(j, j, j)

        zbuf = rows[DEPTH - 1]

        @pl.loop(0, CHUNK)
        def _(r):
            @pl.loop(0, D, step=LANES)
            def _(c0):
                zbuf[r, pl.ds(c0, LANES)] = jnp.zeros((LANES,), jnp.float32)

        my_base = sid * step_rows
        is_last = sid == SUBCORES - 1

        def zero_region(nrows):
            z_full = (nrows // CHUNK) * CHUNK

            @pl.loop(0, z_full, step=CHUNK)
            def _(r0):
                pltpu.sync_copy(zbuf, acc.at[pl.ds(my_base + r0, CHUNK)])

            if nrows % CHUNK:
                pltpu.sync_copy(
                    zbuf.at[pl.ds(0, nrows % CHUNK)],
                    acc.at[pl.ds(my_base + z_full, nrows % CHUNK)])

        if step_rows == last_rows:
            zero_region(step_rows)
        else:
            @pl.when(is_last)
            def _():
                zero_region(last_rows)

            @pl.when(jnp.logical_not(is_last))
            def _():
                zero_region(step_rows)

        plsc.subcore_barrier()

        G = DEPTH - SDEPTH

        def body(k, j):
            wait_gather(k, j % DEPTH, j % IDEPTH)
            wait_scatter(k - SDEPTH, (j - SDEPTH) % DEPTH,
                         (j - SDEPTH) % IDEPTH)
            fire_idx(k + IDEPTH - SDEPTH, (j - SDEPTH) % IDEPTH)
            wait_idx(k + G, (j + G) % IDEPTH)
            fire_gather(k + G, (j + G) % DEPTH, (j + G) % IDEPTH)
            fire_scatter(k, j % DEPTH, j % IDEPTH)

        @pl.loop(0, n_loop, step=IDEPTH)
        def _(k0):
            for j in range(IDEPTH):
                body(k0 + j, j)

        for kd in range(n_loop - SDEPTH, n_loop):
            @pl.when(kd < n_mine)
            def _(kd=kd):
                pltpu.make_async_copy(rows[kd % DEPTH],
                                      acc.at[islots[0].at[1]],
                                      ssems[kd % DEPTH]).wait()

        plsc.subcore_barrier()

        def export_region(nrows):
            pltpu.sync_copy(acc.at[pl.ds(my_base, nrows)],
                            out_hbm.at[cid, pl.ds(my_base, nrows)])

        if step_rows == last_rows:
            export_region(step_rows)
        else:
            @pl.when(is_last)
            def _():
                export_region(last_rows)

            @pl.when(jnp.logical_not(is_last))
            def _():
                export_region(step_rows)

    return agg_kernel(h, ei)


def _tc_mlp(h, parts, W1, b1, W2, b2, scale, beta, layer, n_valid, block,
            with_mean):
    NP, D = h.shape
    grid = NP // block

    def body(h_ref, p_ref, w1_ref, b1_ref, w2_ref, b2_ref, s_ref, t_ref,
             out_ref, mean_ref):
        i = pl.program_id(0)
        z = h_ref[...] + p_ref[0] + p_ref[1]
        z = lax.dot_general(
            z.astype(jnp.bfloat16), w1_ref[0].astype(jnp.bfloat16),
            (((1,), (0,)), ((), ())),
            preferred_element_type=jnp.float32) + b1_ref[0]
        z = jnp.maximum(z, 0.0)
        z = lax.dot_general(
            z.astype(jnp.bfloat16), w2_ref[0].astype(jnp.bfloat16),
            (((1,), (0,)), ((), ())),
            preferred_element_type=jnp.float32) + b2_ref[0]
        z = z * s_ref[0] + t_ref[0]
        hn = jnp.maximum(z, 0.0)
        out_ref[...] = hn
        if mean_ref is not None:
            rid = i * block + lax.broadcasted_iota(jnp.int32, (block, D), 0)
            part = jnp.sum(jnp.where(rid < n_valid, hn, 0.0), axis=0,
                           keepdims=True)

            @pl.when(i == 0)
            def _():
                mean_ref[...] = jnp.zeros_like(mean_ref)

            mean_ref[...] += part

            @pl.when(i == grid - 1)
            def _():
                mean_ref[...] = mean_ref[...] * (1.0 / n_valid)

    if with_mean:
        kern = body
        out_specs = [pl.BlockSpec((block, D), lambda i: (i, 0)),
                     pl.BlockSpec((1, D), lambda i: (0, 0))]
        out_shape = [jax.ShapeDtypeStruct((NP, D), jnp.float32),
                     jax.ShapeDtypeStruct((1, D), jnp.float32)]
    else:
        def kern(*refs):
            body(*refs, None)
        out_specs = [pl.BlockSpec((block, D), lambda i: (i, 0))]
        out_shape = [jax.ShapeDtypeStruct((NP, D), jnp.float32)]

    li = layer

    res = pl.pallas_call(
        kern,
        grid=(grid,),
        in_specs=[
            pl.BlockSpec((block, D), lambda i: (i, 0)),
            pl.BlockSpec((NUM_SC, block, D), lambda i: (0, i, 0)),
            pl.BlockSpec((1, D, D), lambda i: (li, 0, 0)),
            pl.BlockSpec((1, 1, D), lambda i: (li, 0, 0)),
            pl.BlockSpec((1, D, D), lambda i: (li, 0, 0)),
            pl.BlockSpec((1, 1, D), lambda i: (li, 0, 0)),
            pl.BlockSpec((1, 1, D), lambda i: (li, 0, 0)),
            pl.BlockSpec((1, 1, D), lambda i: (li, 0, 0)),
        ],
        out_specs=out_specs,
        out_shape=out_shape,
    )(h, parts, W1, b1, W2, b2, scale, beta)
    return res if with_mean else (res[0], None)


def kernel(x, edge_index, W1, b1, W2, b2, gamma, beta):
    N, D = x.shape
    E = edge_index.shape[1]
    L = W1.shape[0]
    rem = E % CHUNK
    if rem:
        dummy = (jnp.arange(CHUNK - rem, dtype=jnp.int32) % PAD_ROWS) + N
        ei = jnp.concatenate(
            [edge_index, jnp.tile(dummy, (2, 1))], axis=1)
        NP = -(-(N + PAD_ROWS) // SUBCORES) * SUBCORES
    else:
        ei = edge_index
        NP = -(-N // SUBCORES) * SUBCORES
    n_real_chunks = ei.shape[1] // CHUNK
    n_full = n_real_chunks // NUM_TILES
    n_extra = n_real_chunks - n_full * NUM_TILES

    block = next(b for b in (2048, 2000, 1264, 1024, 512, 256, 128, 64,
                             32, 16, 8)
                 if NP % b == 0)

    if NP == N:
        h = x.astype(jnp.float32)
    else:
        h = jnp.zeros((NP, D), jnp.float32).at[:N].set(x.astype(jnp.float32))
    inv_std = 1.0 / jnp.sqrt(1.0 + BN_EPS_CONST)
    scales = (gamma * inv_std).astype(jnp.float32).reshape(L, 1, D)
    b1r = b1.reshape(L, 1, D)
    b2r = b2.reshape(L, 1, D)
    betar = beta.reshape(L, 1, D)

    mean = None
    for i in range(L):
        parts = _sc_aggregate(h, ei, n_full, n_extra)
        h, mean = _tc_mlp(h, parts, W1, b1r, W2, b2r, scales, betar, i, N,
                          block, with_mean=(i == L - 1))
    return mean


# speedup vs baseline: 1.0760x; 1.0760x over previous
import functools

import jax
import jax.numpy as jnp
from jax import lax
from jax.experimental import pallas as pl
from jax.experimental.pallas import tpu as pltpu
from jax.experimental.pallas import tpu_sc as plsc

NUM_SC = 2
SUBCORES = 16
NUM_TILES = NUM_SC * SUBCORES
CHUNK = 128
DEPTH = 3
IDEPTH = 6
SDEPTH = 1
LANES = 16
PAD_ROWS = 16
BN_EPS_CONST = 1e-5


def _sc_aggregate(h, ei, n_full, n_extra):
    NP, D = h.shape
    step_rows = -(-NP // (SUBCORES * 8)) * 8
    last_rows = NP - (SUBCORES - 1) * step_rows
    assert last_rows > 0 and last_rows % 8 == 0
    n_loop = -(-(n_full + (1 if n_extra else 0)) // IDEPTH) * IDEPTH
    mesh = plsc.VectorSubcoreMesh(
        core_axis_name="c", subcore_axis_name="s",
        num_cores=NUM_SC, num_subcores=SUBCORES)

    @functools.partial(
        pl.kernel,
        out_type=jax.ShapeDtypeStruct((NUM_SC, NP, D), jnp.float32),
        mesh=mesh,
        scratch_types=[
            [pltpu.VMEM((2, CHUNK), jnp.int32) for _ in range(IDEPTH)],
            [pltpu.VMEM((CHUNK, D), jnp.float32) for _ in range(DEPTH)],
            pltpu.VMEM_SHARED((NP, D), jnp.float32),
            [pltpu.SemaphoreType.DMA for _ in range(IDEPTH)],
            [pltpu.SemaphoreType.DMA for _ in range(DEPTH)],
            [pltpu.SemaphoreType.DMA for _ in range(DEPTH)],
        ],
    )
    def agg_kernel(h_hbm, ei_hbm, out_hbm, islots, rows, acc,
                   isems, gsems, ssems):
        cid = lax.axis_index("c")
        sid = lax.axis_index("s")
        wid = sid * NUM_SC + cid
        n_mine = n_full + jnp.where(wid < n_extra, 1, 0)

        def _guard(k, lo_ok=True):
            return (k < n_mine) if lo_ok else ((k >= 0) & (k < n_mine))

        def fire_idx(k, si):
            @pl.when(_guard(k))
            def _():
                off = (k * NUM_TILES + wid) * CHUNK
                pltpu.async_copy(ei_hbm.at[0, pl.ds(off, CHUNK)],
                                 islots[si].at[0], isems[si])
                pltpu.async_copy(ei_hbm.at[1, pl.ds(off, CHUNK)],
                                 islots[si].at[1], isems[si])

        def wait_idx(k, si):
            @pl.when(_guard(k))
            def _():
                pltpu.make_async_copy(ei_hbm.at[0, pl.ds(0, CHUNK)],
                                      islots[si].at[0], isems[si]).wait()
                pltpu.make_async_copy(ei_hbm.at[1, pl.ds(0, CHUNK)],
                                      islots[si].at[1], isems[si]).wait()

        def fire_gather(k, s, si):
            @pl.when(_guard(k))
            def _():
                pltpu.async_copy(h_hbm.at[islots[si].at[0]], rows[s],
                                 gsems[s])

        def wait_gather(k, s, si):
            @pl.when(_guard(k))
            def _():
                pltpu.make_async_copy(
                    h_hbm.at[islots[si].at[0]], rows[s], gsems[s]).wait()

        def fire_scatter(k, s, si):
            @pl.when(_guard(k))
            def _():
                pltpu.async_copy(rows[s], acc.at[islots[si].at[1]],
                                 ssems[s], add=True)

        def wait_scatter(k, s, si):
            @pl.when(_guard(k, lo_ok=False))
            def _():
                pltpu.make_async_copy(rows[s], acc.at[islots[si].at[1]],
                                      ssems[s]).wait()

        for j in range(IDEPTH - SDEPTH):
            fire_idx(j, j)
        for j in range(DEPTH - SDEPTH):
            wait_idx(j, j)
            fire_gather(j, j, j)

        zbuf = rows[DEPTH - 1]

        @pl.loop(0, CHUNK)
        def _(r):
            @pl.loop(0, D, step=LANES)
            def _(c0):
                zbuf[r, pl.ds(c0, LANES)] = jnp.zeros((LANES,), jnp.float32)

        my_base = sid * step_rows
        is_last = sid == SUBCORES - 1

        def zero_region(nrows):
            z_full = (nrows // CHUNK) * CHUNK

            @pl.loop(0, z_full, step=CHUNK)
            def _(r0):
                pltpu.sync_copy(zbuf, acc.at[pl.ds(my_base + r0, CHUNK)])

            if nrows % CHUNK:
                pltpu.sync_copy(
                    zbuf.at[pl.ds(0, nrows % CHUNK)],
                    acc.at[pl.ds(my_base + z_full, nrows % CHUNK)])

        if step_rows == last_rows:
            zero_region(step_rows)
        else:
            @pl.when(is_last)
            def _():
                zero_region(last_rows)

            @pl.when(jnp.logical_not(is_last))
            def _():
                zero_region(step_rows)

        plsc.subcore_barrier()

        G = DEPTH - SDEPTH

        def body(k, j):
            wait_gather(k, j % DEPTH, j % IDEPTH)
            wait_scatter(k - SDEPTH, (j - SDEPTH) % DEPTH,
                         (j - SDEPTH) % IDEPTH)
            fire_idx(k + IDEPTH - SDEPTH, (j - SDEPTH) % IDEPTH)
            wait_idx(k + G, (j + G) % IDEPTH)
            fire_gather(k + G, (j + G) % DEPTH, (j + G) % IDEPTH)
            fire_scatter(k, j % DEPTH, j % IDEPTH)

        @pl.loop(0, n_loop, step=IDEPTH)
        def _(k0):
            for j in range(IDEPTH):
                body(k0 + j, j)

        for kd in range(n_loop - SDEPTH, n_loop):
            @pl.when(kd < n_mine)
            def _(kd=kd):
                pltpu.make_async_copy(rows[kd % DEPTH],
                                      acc.at[islots[0].at[1]],
                                      ssems[kd % DEPTH]).wait()

        plsc.subcore_barrier()

        def export_region(nrows):
            pltpu.sync_copy(acc.at[pl.ds(my_base, nrows)],
                            out_hbm.at[cid, pl.ds(my_base, nrows)])

        if step_rows == last_rows:
            export_region(step_rows)
        else:
            @pl.when(is_last)
            def _():
                export_region(last_rows)

            @pl.when(jnp.logical_not(is_last))
            def _():
                export_region(step_rows)

    return agg_kernel(h, ei)


def _tc_mlp(h, parts, W1, b1, W2, b2, scale, beta, layer, n_valid, block,
            with_mean):
    NP, D = h.shape
    grid = NP // block

    def body(h_ref, p_ref, w1_ref, b1_ref, w2_ref, b2_ref, s_ref, t_ref,
             out_ref, mean_ref):
        i = pl.program_id(0)
        z = h_ref[...] + p_ref[0] + p_ref[1]
        z = lax.dot_general(
            z.astype(jnp.bfloat16), w1_ref[0].astype(jnp.bfloat16),
            (((1,), (0,)), ((), ())),
            preferred_element_type=jnp.float32) + b1_ref[0]
        z = jnp.maximum(z, 0.0)
        z = lax.dot_general(
            z.astype(jnp.bfloat16), w2_ref[0].astype(jnp.bfloat16),
            (((1,), (0,)), ((), ())),
            preferred_element_type=jnp.float32) + b2_ref[0]
        z = z * s_ref[0] + t_ref[0]
        hn = jnp.maximum(z, 0.0)
        out_ref[...] = hn
        if mean_ref is not None:
            rid = i * block + lax.broadcasted_iota(jnp.int32, (block, D), 0)
            part = jnp.sum(jnp.where(rid < n_valid, hn, 0.0), axis=0,
                           keepdims=True)

            @pl.when(i == 0)
            def _():
                mean_ref[...] = jnp.zeros_like(mean_ref)

            mean_ref[...] += part

            @pl.when(i == grid - 1)
            def _():
                mean_ref[...] = mean_ref[...] * (1.0 / n_valid)

    if with_mean:
        kern = body
        out_specs = [pl.BlockSpec((block, D), lambda i: (i, 0)),
                     pl.BlockSpec((1, D), lambda i: (0, 0))]
        out_shape = [jax.ShapeDtypeStruct((NP, D), jnp.float32),
                     jax.ShapeDtypeStruct((1, D), jnp.float32)]
    else:
        def kern(*refs):
            body(*refs, None)
        out_specs = [pl.BlockSpec((block, D), lambda i: (i, 0))]
        out_shape = [jax.ShapeDtypeStruct((NP, D), jnp.float32)]

    li = layer

    res = pl.pallas_call(
        kern,
        grid=(grid,),
        in_specs=[
            pl.BlockSpec((block, D), lambda i: (i, 0)),
            pl.BlockSpec((NUM_SC, block, D), lambda i: (0, i, 0)),
            pl.BlockSpec((1, D, D), lambda i: (li, 0, 0)),
            pl.BlockSpec((1, 1, D), lambda i: (li, 0, 0)),
            pl.BlockSpec((1, D, D), lambda i: (li, 0, 0)),
            pl.BlockSpec((1, 1, D), lambda i: (li, 0, 0)),
            pl.BlockSpec((1, 1, D), lambda i: (li, 0, 0)),
            pl.BlockSpec((1, 1, D), lambda i: (li, 0, 0)),
        ],
        out_specs=out_specs,
        out_shape=out_shape,
    )(h, parts, W1, b1, W2, b2, scale, beta)
    return res if with_mean else (res[0], None)


def kernel(x, edge_index, W1, b1, W2, b2, gamma, beta):
    N, D = x.shape
    E = edge_index.shape[1]
    L = W1.shape[0]
    rem = E % CHUNK
    if rem:
        dummy = (jnp.arange(CHUNK - rem, dtype=jnp.int32) % PAD_ROWS) + N
        ei = jnp.concatenate(
            [edge_index, jnp.tile(dummy, (2, 1))], axis=1)
        NP = -(-(N + PAD_ROWS) // SUBCORES) * SUBCORES
    else:
        ei = edge_index
        NP = -(-N // SUBCORES) * SUBCORES
    n_real_chunks = ei.shape[1] // CHUNK
    n_full = n_real_chunks // NUM_TILES
    n_extra = n_real_chunks - n_full * NUM_TILES

    block = next(b for b in (2048, 2000, 1264, 1024, 512, 256, 128, 64,
                             32, 16, 8)
                 if NP % b == 0)

    if NP == N:
        h = x.astype(jnp.float32)
    else:
        h = jnp.zeros((NP, D), jnp.float32).at[:N].set(x.astype(jnp.float32))
    inv_std = 1.0 / jnp.sqrt(1.0 + BN_EPS_CONST)
    scales = (gamma * inv_std).astype(jnp.float32).reshape(L, 1, D)
    b1r = b1.reshape(L, 1, D)
    b2r = b2.reshape(L, 1, D)
    betar = beta.reshape(L, 1, D)

    mean = None
    for i in range(L):
        parts = _sc_aggregate(h, ei, n_full, n_extra)
        h, mean = _tc_mlp(h, parts, W1, b1r, W2, b2r, scales, betar, i, N,
                          block, with_mean=(i == L - 1))
    return mean
